# Initial kernel scaffold; baseline (speedup 1.0000x reference)
import jax
import jax.numpy as jnp
from jax.experimental import pallas as pl


def kernel(x, W1, b1, W2, b2, t1, t2):
    raise NotImplementedError("write your pallas kernel here")


# speedup vs baseline: 1.0307x; 1.0307x over previous
import math

import jax
import jax.numpy as jnp
from jax.experimental import pallas as pl
from jax.experimental.pallas import tpu as pltpu

_NUM_EXPERTS = 8
_N_TOK = 8192
_TOK_PER_EXPERT = _N_TOK // _NUM_EXPERTS
_TB = 256
_TILES_PER_EXPERT = _TOK_PER_EXPERT // _TB
_CLAMP_MAX = math.log(100.0)


def _moe_body(t1_ref, t2_ref, x_ref, w1_ref, b1_ref, w2_ref, b2_ref, o_ref):
    s1 = jnp.exp(jnp.minimum(t1_ref[0], _CLAMP_MAX))
    s2 = jnp.exp(jnp.minimum(t2_ref[0], _CLAMP_MAX))
    h = jnp.dot(x_ref[...], w1_ref[0], preferred_element_type=jnp.float32)
    h = (h + b1_ref[0]) * s1
    h = jnp.maximum(h, 0.0).astype(jnp.bfloat16)
    o = jnp.dot(h, w2_ref[0], preferred_element_type=jnp.float32)
    o_ref[...] = (o + b2_ref[0]) * s2


def kernel(x, W1, b1, W2, b2, t1, t2):
    d_in = x.shape[1]
    d_hid = W1.shape[2]
    d_out = W2.shape[2]
    xb = x.astype(jnp.bfloat16)
    w1b = W1.astype(jnp.bfloat16)
    w2b = W2.astype(jnp.bfloat16)
    b1r = b1.reshape(_NUM_EXPERTS, 1, d_hid)
    b2r = b2.reshape(_NUM_EXPERTS, 1, d_out)

    grid = (_NUM_EXPERTS, _TILES_PER_EXPERT)
    return pl.pallas_call(
        _moe_body,
        grid=grid,
        in_specs=[
            pl.BlockSpec(memory_space=pltpu.SMEM),
            pl.BlockSpec(memory_space=pltpu.SMEM),
            pl.BlockSpec((_TB, d_in), lambda e, t: (e * _TILES_PER_EXPERT + t, 0)),
            pl.BlockSpec((1, d_in, d_hid), lambda e, t: (e, 0, 0)),
            pl.BlockSpec((1, 1, d_hid), lambda e, t: (e, 0, 0)),
            pl.BlockSpec((1, d_hid, d_out), lambda e, t: (e, 0, 0)),
            pl.BlockSpec((1, 1, d_out), lambda e, t: (e, 0, 0)),
        ],
        out_specs=pl.BlockSpec(
            (_TB, d_out), lambda e, t: (e * _TILES_PER_EXPERT + t, 0)
        ),
        out_shape=jax.ShapeDtypeStruct((_N_TOK, d_out), jnp.float32),
        compiler_params=pltpu.CompilerParams(
            dimension_semantics=("arbitrary", "arbitrary"),
        ),
    )(t1, t2, xb, w1b, b1r, w2b, b2r)


# speedup vs baseline: 1.5343x; 1.4886x over previous
import math

import jax
import jax.numpy as jnp
from jax.experimental import pallas as pl
from jax.experimental.pallas import tpu as pltpu

_NUM_EXPERTS = 8
_N_TOK = 8192
_TOK_PER_EXPERT = _N_TOK // _NUM_EXPERTS
_TB = 512
_TILES_PER_EXPERT = _TOK_PER_EXPERT // _TB
_CLAMP_MAX = math.log(100.0)


def _layer1_body(t_ref, x_ref, w_ref, b_ref, o_ref):
    s = jnp.exp(jnp.minimum(t_ref[0], _CLAMP_MAX))
    h = jnp.dot(x_ref[...], w_ref[0], preferred_element_type=jnp.float32)
    h = (h + b_ref[0]) * s
    o_ref[...] = jnp.maximum(h, 0.0).astype(jnp.bfloat16)


def _layer2_body(t_ref, x_ref, w_ref, b_ref, o_ref):
    s = jnp.exp(jnp.minimum(t_ref[0], _CLAMP_MAX))
    o = jnp.dot(x_ref[...], w_ref[0], preferred_element_type=jnp.float32)
    o_ref[...] = (o + b_ref[0]) * s


def _layer_call(body, x, w, b, t, out_dtype):
    d_in = x.shape[1]
    d_out = w.shape[2]
    br = b.reshape(_NUM_EXPERTS, 1, d_out)
    grid = (_NUM_EXPERTS, _TILES_PER_EXPERT)
    return pl.pallas_call(
        body,
        grid=grid,
        in_specs=[
            pl.BlockSpec(memory_space=pltpu.SMEM),
            pl.BlockSpec((_TB, d_in), lambda e, i: (e * _TILES_PER_EXPERT + i, 0)),
            pl.BlockSpec((1, d_in, d_out), lambda e, i: (e, 0, 0)),
            pl.BlockSpec((1, 1, d_out), lambda e, i: (e, 0, 0)),
        ],
        out_specs=pl.BlockSpec(
            (_TB, d_out), lambda e, i: (e * _TILES_PER_EXPERT + i, 0)
        ),
        out_shape=jax.ShapeDtypeStruct((_N_TOK, d_out), out_dtype),
        compiler_params=pltpu.CompilerParams(
            dimension_semantics=("arbitrary", "arbitrary"),
        ),
    )(t, x, w, br)


def kernel(x, W1, b1, W2, b2, t1, t2):
    h = _layer_call(_layer1_body, x, W1, b1, t1, jnp.bfloat16)
    return _layer_call(_layer2_body, h, W2, b2, t2, jnp.float32)


# speedup vs baseline: 1.7758x; 1.1574x over previous
import math

import jax
import jax.numpy as jnp
from jax.experimental import pallas as pl
from jax.experimental.pallas import tpu as pltpu

_NUM_EXPERTS = 8
_N_TOK = 8192
_TB = _N_TOK // _NUM_EXPERTS
_NSPLIT = 2
_CLAMP_MAX = math.log(100.0)


def _layer1_body(t_ref, x_ref, w_ref, b_ref, o_ref):
    s = jnp.exp(jnp.minimum(t_ref[0], _CLAMP_MAX))
    h = jnp.dot(x_ref[...], w_ref[0], preferred_element_type=jnp.float32)
    h = (h + b_ref[0]) * s
    o_ref[...] = jnp.maximum(h, 0.0).astype(jnp.bfloat16)


def _layer2_body(t_ref, x_ref, w_ref, b_ref, o_ref):
    s = jnp.exp(jnp.minimum(t_ref[0], _CLAMP_MAX))
    o = jnp.dot(x_ref[...], w_ref[0], preferred_element_type=jnp.float32)
    o_ref[...] = (o + b_ref[0]) * s


def _layer_call(body, x, w, b, t, out_dtype):
    d_in = x.shape[1]
    d_out = w.shape[2]
    dcol = d_out // _NSPLIT
    br = b.reshape(_NUM_EXPERTS, 1, d_out)
    grid = (_NUM_EXPERTS, _NSPLIT)
    return pl.pallas_call(
        body,
        grid=grid,
        in_specs=[
            pl.BlockSpec(memory_space=pltpu.SMEM),
            pl.BlockSpec((_TB, d_in), lambda e, j: (e, 0)),
            pl.BlockSpec((1, d_in, dcol), lambda e, j: (e, 0, j)),
            pl.BlockSpec((1, 1, dcol), lambda e, j: (e, 0, j)),
        ],
        out_specs=pl.BlockSpec((_TB, dcol), lambda e, j: (e, j)),
        out_shape=jax.ShapeDtypeStruct((_N_TOK, d_out), out_dtype),
        compiler_params=pltpu.CompilerParams(
            dimension_semantics=("arbitrary", "arbitrary"),
        ),
    )(t, x, w, br)


def kernel(x, W1, b1, W2, b2, t1, t2):
    h = _layer_call(_layer1_body, x, W1, b1, t1, jnp.bfloat16)
    return _layer_call(_layer2_body, h, W2, b2, t2, jnp.float32)
